# baseline (speedup 1.0000x reference)
import jax
import jax.numpy as jnp
from jax import lax
from jax.experimental import pallas as pl
from jax.experimental.pallas import tpu as pltpu

N = 64
B = 256
F = 256
ITERS = 10
DIL = (N - 8) // 2


def _mlp_kernel(x_aug_ref, w1b_ref, w2b_ref, h2t_ref):
    h1t = lax.dot_general(
        w1b_ref[...], x_aug_ref[...],
        (((1,), (1,)), ((), ())),
        preferred_element_type=jnp.float32)
    h1t_aug = jnp.concatenate(
        [h1t, jnp.ones((1, B), jnp.float32)], axis=0)
    h2t_ref[...] = lax.dot_general(
        w2b_ref[...], h1t_aug,
        (((1,), (0,)), ((), ())),
        preferred_element_type=jnp.float32)


def _mlp(x, w1, b1, w2, b2):
    x_aug = jnp.concatenate([x, jnp.ones((B, 1), jnp.float32)], axis=1)
    w1b = jnp.concatenate([w1, b1[:, None]], axis=1)
    w2b = jnp.concatenate([w2, b2[:, None]], axis=1)
    n_out = w2.shape[0]
    blk = n_out // 2
    return pl.pallas_call(
        _mlp_kernel,
        grid=(2,),
        in_specs=[
            pl.BlockSpec((B, F + 1), lambda i: (0, 0)),
            pl.BlockSpec((2 * N, F + 1), lambda i: (0, 0)),
            pl.BlockSpec((blk, 2 * N + 1), lambda i: (i, 0)),
        ],
        out_specs=pl.BlockSpec((blk, B), lambda i: (i, 0)),
        out_shape=jax.ShapeDtypeStruct((n_out, B), jnp.float32),
        compiler_params=pltpu.CompilerParams(
            dimension_semantics=("parallel",)),
    )(x_aug, w1b, w2b)


def _big_mm_kernel(h2t_ref, w3_ref, b3_ref, out_ref):
    acc = lax.dot_general(
        h2t_ref[...], w3_ref[...],
        (((0,), (1,)), ((), ())),
        preferred_element_type=jnp.float32)
    out_ref[...] = acc + b3_ref[...]


def _big_mm(h2t, w3, b3):
    m, k = w3.shape
    cols = 256
    grid = m // cols
    return pl.pallas_call(
        _big_mm_kernel,
        grid=(grid,),
        in_specs=[
            pl.BlockSpec((k, B), lambda i: (0, 0)),
            pl.BlockSpec((cols, k), lambda i: (i, 0)),
            pl.BlockSpec((1, cols), lambda i: (0, i)),
        ],
        out_specs=pl.BlockSpec((B, cols), lambda i: (0, i)),
        out_shape=jax.ShapeDtypeStruct((B, m), jnp.float32),
        compiler_params=pltpu.CompilerParams(
            dimension_semantics=("parallel",)),
    )(h2t, w3, b3[None, :])


BT = 8

_L = (72, 70, 68, 66, 64)
_D = (DIL, 1, 1, 1, 1)


def _conv_layer(xb, wk_aug, wk1, wk2, out_len, dil):
    x0 = jnp.concatenate(
        [xb[:, 0:out_len], jnp.ones((1, out_len), jnp.float32)], axis=0)
    y = lax.dot_general(wk_aug, x0, (((1,), (0,)), ((), ())),
                        preferred_element_type=jnp.float32)
    y = y + lax.dot_general(wk1, xb[:, dil:dil + out_len],
                            (((1,), (0,)), ((), ())),
                            preferred_element_type=jnp.float32)
    y = y + lax.dot_general(wk2, xb[:, 2 * dil:2 * dil + out_len],
                            (((1,), (0,)), ((), ())),
                            preferred_element_type=jnp.float32)
    return y


def _conv_sink_kernel(h_ref, c1a_ref, c11_ref, c12_ref, c2a_ref, c21_ref,
                      c22_ref, c3a_ref, c31_ref, c32_ref, c4a_ref, c41_ref,
                      c42_ref, c5a_ref, c51_ref, c52_ref, out_ref):
    cs = [(c1a_ref, c11_ref, c12_ref), (c2a_ref, c21_ref, c22_ref),
          (c3a_ref, c31_ref, c32_ref), (c4a_ref, c41_ref, c42_ref),
          (c5a_ref, c51_ref, c52_ref)]
    ys = []
    for b in range(BT):
        y = h_ref[b]
        for li, (ca, c1, c2) in enumerate(cs):
            y = _conv_layer(y, ca[...], c1[...], c2[...], _L[li], _D[li])
        ys.append(y)
    o = jnp.stack(ys, axis=0)
    o = o * o + 0.001

    def _sink_iter(_, o):
        o = o / jnp.sum(o, axis=2, keepdims=True)
        o = o / jnp.sum(o, axis=1, keepdims=True)
        return o

    out_ref[...] = lax.fori_loop(0, ITERS, _sink_iter, o)


def _make_const_spec(w):
    nd = w.ndim
    return pl.BlockSpec(w.shape, lambda i, _n=nd: (0,) * _n)


def _conv_sink(h3, flat_ws):
    w_specs = [_make_const_spec(w) for w in flat_ws]
    return pl.pallas_call(
        _conv_sink_kernel,
        grid=(B // BT,),
        in_specs=[pl.BlockSpec((BT, 2 * N, 2 * N), lambda i: (i, 0, 0))]
        + w_specs,
        out_specs=pl.BlockSpec((BT, N, N), lambda i: (i, 0, 0)),
        out_shape=jax.ShapeDtypeStruct((B, N, N), jnp.float32),
        compiler_params=pltpu.CompilerParams(
            dimension_semantics=("parallel",)),
    )(h3, *flat_ws)


def _prep_conv_weights(cw, cb):
    wk0 = jnp.concatenate([cw[:, :, 0], cb[:, None]], axis=1)
    return (wk0, cw[:, :, 1], cw[:, :, 2])


def kernel(x, w1, b1, w2, b2, w3, b3,
           cw1, cb1, cw2, cb2, cw3, cb3, cw4, cb4, cw5, cb5):
    h2t = _mlp(x, w1, b1, w2, b2)
    h3 = _big_mm(h2t, w3, b3)
    h3r = h3.reshape(B, 2 * N, 2 * N)
    flat_ws = []
    for cw, cb in ((cw1, cb1), (cw2, cb2), (cw3, cb3), (cw4, cb4),
                   (cw5, cb5)):
        flat_ws.extend(_prep_conv_weights(cw, cb))
    return _conv_sink(h3r, flat_ws)


# speedup vs baseline: 1.9448x; 1.9448x over previous
import jax
import jax.numpy as jnp
from jax import lax
from jax.experimental import pallas as pl
from jax.experimental.pallas import tpu as pltpu

N = 64
B = 256
F = 256
ITERS = 10
DIL = (N - 8) // 2


def _mlp_kernel(x_aug_ref, w1b_ref, w2b_ref, h2t_ref):
    h1t = lax.dot_general(
        w1b_ref[...], x_aug_ref[...],
        (((1,), (1,)), ((), ())),
        preferred_element_type=jnp.float32)
    h1t_aug = jnp.concatenate(
        [h1t, jnp.ones((1, B), jnp.float32)], axis=0)
    h2t_ref[...] = lax.dot_general(
        w2b_ref[...], h1t_aug,
        (((1,), (0,)), ((), ())),
        preferred_element_type=jnp.float32)


def _mlp(x, w1, b1, w2, b2):
    x_aug = jnp.concatenate([x, jnp.ones((B, 1), jnp.float32)], axis=1)
    w1b = jnp.concatenate([w1, b1[:, None]], axis=1)
    w2b = jnp.concatenate([w2, b2[:, None]], axis=1)
    n_out = w2.shape[0]
    blk = n_out // 2
    return pl.pallas_call(
        _mlp_kernel,
        grid=(2,),
        in_specs=[
            pl.BlockSpec((B, F + 1), lambda i: (0, 0)),
            pl.BlockSpec((2 * N, F + 1), lambda i: (0, 0)),
            pl.BlockSpec((blk, 2 * N + 1), lambda i: (i, 0)),
        ],
        out_specs=pl.BlockSpec((blk, B), lambda i: (i, 0)),
        out_shape=jax.ShapeDtypeStruct((n_out, B), jnp.float32),
        compiler_params=pltpu.CompilerParams(
            dimension_semantics=("arbitrary",)),
    )(x_aug, w1b, w2b)


def _big_mm_kernel(h2t_ref, w3_ref, b3_ref, out_ref):
    acc = lax.dot_general(
        h2t_ref[...], w3_ref[...],
        (((0,), (1,)), ((), ())),
        preferred_element_type=jnp.float32)
    out_ref[...] = acc + b3_ref[...]


def _big_mm(h2t, w3, b3):
    m, k = w3.shape
    cols = 256
    grid = m // cols
    return pl.pallas_call(
        _big_mm_kernel,
        grid=(grid,),
        in_specs=[
            pl.BlockSpec((k, B), lambda i: (0, 0)),
            pl.BlockSpec((cols, k), lambda i: (i, 0)),
            pl.BlockSpec((1, cols), lambda i: (0, i)),
        ],
        out_specs=pl.BlockSpec((B, cols), lambda i: (0, i)),
        out_shape=jax.ShapeDtypeStruct((B, m), jnp.float32),
        compiler_params=pltpu.CompilerParams(
            dimension_semantics=("arbitrary",)),
    )(h2t, w3, b3[None, :])


BT = 16


def _shift_rows(x, k):
    if k == 0:
        return x
    return jnp.concatenate([x[k:], x[:k]], axis=0)


def _conv_stack(x, wt0, wt1, wt2, bias, dil):
    y = lax.dot_general(x, wt0, (((1,), (0,)), ((), ())),
                        preferred_element_type=jnp.float32)
    y = y + lax.dot_general(_shift_rows(x, dil), wt1,
                            (((1,), (0,)), ((), ())),
                            preferred_element_type=jnp.float32)
    y = y + lax.dot_general(_shift_rows(x, 2 * dil), wt2,
                            (((1,), (0,)), ((), ())),
                            preferred_element_type=jnp.float32)
    return y + bias


def _sinkhorn(o):
    for _ in range(ITERS):
        o = o / jnp.sum(o, axis=1, keepdims=True)
        o = o / jnp.sum(o, axis=2, keepdims=True)
    return o


def _conv_sink_kernel(h_ref, w10, w11, w12, bb1, w20, w21, w22, bb2,
                      w30, w31, w32, bb3, w40, w41, w42, bb4,
                      w50, w51, w52, bb5, out_ref):
    x = jnp.swapaxes(h_ref[...], 1, 2).reshape(BT * 2 * N, 2 * N)
    y = _conv_stack(x, w10[...], w11[...], w12[...], bb1[...], DIL)
    y = y.reshape(BT, 2 * N, 3 * N)[:, :72, :].reshape(BT * 72, 3 * N)
    y = _conv_stack(y, w20[...], w21[...], w22[...], bb2[...], 1)
    y = _conv_stack(y, w30[...], w31[...], w32[...], bb3[...], 1)
    y = _conv_stack(y, w40[...], w41[...], w42[...], bb4[...], 1)
    y = _conv_stack(y, w50[...], w51[...], w52[...], bb5[...], 1)
    o = y.reshape(BT, 72, N)[:, :N, :]
    o = o * o + 0.001
    q = BT // 2
    parts = [_sinkhorn(o[i * q:(i + 1) * q]) for i in range(2)]
    o = jnp.concatenate(parts, axis=0)
    out_ref[...] = jnp.swapaxes(o, 1, 2)


def _make_const_spec(w):
    nd = w.ndim
    return pl.BlockSpec(w.shape, lambda i, _n=nd: (0,) * _n)


def _conv_sink(h3, flat_ws):
    w_specs = [_make_const_spec(w) for w in flat_ws]
    return pl.pallas_call(
        _conv_sink_kernel,
        grid=(B // BT,),
        in_specs=[pl.BlockSpec((BT, 2 * N, 2 * N), lambda i: (i, 0, 0))]
        + w_specs,
        out_specs=pl.BlockSpec((BT, N, N), lambda i: (i, 0, 0)),
        out_shape=jax.ShapeDtypeStruct((B, N, N), jnp.float32),
        compiler_params=pltpu.CompilerParams(
            dimension_semantics=("arbitrary",)),
    )(h3, *flat_ws)


def _prep_conv_weights(cw, cb):
    return (cw[:, :, 0].T, cw[:, :, 1].T, cw[:, :, 2].T, cb[None, :])


def kernel(x, w1, b1, w2, b2, w3, b3,
           cw1, cb1, cw2, cb2, cw3, cb3, cw4, cb4, cw5, cb5):
    h2t = _mlp(x, w1, b1, w2, b2)
    h3 = _big_mm(h2t, w3, b3)
    h3r = h3.reshape(B, 2 * N, 2 * N)
    flat_ws = []
    for cw, cb in ((cw1, cb1), (cw2, cb2), (cw3, cb3), (cw4, cb4),
                   (cw5, cb5)):
        flat_ws.extend(_prep_conv_weights(cw, cb))
    return _conv_sink(h3r, flat_ws)
